# baseline (speedup 1.0000x reference)
import functools
import itertools

import jax
import jax.numpy as jnp
from jax import lax
from jax.experimental import pallas as pl
from jax.experimental.pallas import tpu as pltpu
from jax.experimental.pallas import tpu_sc as plsc

_F32 = jnp.float32
_HI = lax.Precision.HIGHEST


def _mm(a, b, prec=None):
    return lax.dot_general(
        a, b, dimension_numbers=(((a.ndim - 1,), (0,)), ((), ())),
        preferred_element_type=_F32, precision=prec)


def _mmt(a, b):
    tb = jnp.bfloat16
    return _mm(a.astype(tb).astype(_F32), b.astype(tb).astype(_F32))


def _mm4(a, b):
    tb = jnp.bfloat16
    ah = a.astype(tb).astype(_F32)
    al = (a - ah).astype(tb).astype(_F32)
    bh = b.astype(tb).astype(_F32)
    bl = (b - bh).astype(tb).astype(_F32)
    return _mm(ah, bh) + (_mm(ah, bl) + (_mm(al, bh) + _mm(al, bl)))


def _swish(x):
    return x * jax.nn.sigmoid(x)


def _e0_body(x_ref, a_ref, b_ref, o_ref):
    c = pl.program_id(0)
    def tap(kd, acc):
        for kh in range(3):
            xs = x_ref[pl.ds(kd + 16 * c, 16), kh:kh + 64, :].reshape(1024, 66)
            acc = acc + _mmt(xs, a_ref[3 * kd + kh])
        return acc
    acc = lax.fori_loop(0, 3, tap, jnp.zeros((1024, 2048), _F32))
    o_ref[...] = _swish(acc + b_ref[...])


def _e0(xp, a9, b2048):
    return pl.pallas_call(
        _e0_body,
        grid=(4,),
        in_specs=[
            pl.BlockSpec((66, 66, 66), lambda c: (0, 0, 0)),
            pl.BlockSpec((9, 66, 2048), lambda c: (0, 0, 0)),
            pl.BlockSpec((1, 2048), lambda c: (0, 0)),
        ],
        out_specs=pl.BlockSpec((1024, 2048), lambda c: (c, 0)),
        out_shape=jax.ShapeDtypeStruct((4096, 2048), _F32),
    )(xp, a9, b2048)


_TAPMAP = {0: (0, 0), 1: (1, 0), 2: (0, 1)}


def _s2_body(p_ref, w_ref, b_ref, o_ref, *, do, s, cin, cout, dchunk):
    for c0 in range(0, do, dchunk):
        cols = []
        for kd in range(3):
            pd, td = _TAPMAP[kd]
            for kh in range(3):
                ph, th = _TAPMAP[kh]
                for kw in range(3):
                    pw, tw = _TAPMAP[kw]
                    pc = (pd * 2 + ph) * 2 + pw
                    cols.append(
                        p_ref[td + c0:td + c0 + dchunk, th:th + s,
                              tw:tw + s, pc * cin:(pc + 1) * cin]
                        .reshape(dchunk * s * s, cin))
        xcat = jnp.concatenate(cols, axis=1)
        acc = _mmt(xcat, w_ref[...])
        o_ref[c0:c0 + dchunk] = _swish(acc + b_ref[...]).reshape(
            dchunk, s, s, cout)


def _s2_conv(p, w27, b, do, s, cin, cout, dchunk):
    body = functools.partial(_s2_body, do=do, s=s, cin=cin, cout=cout,
                             dchunk=dchunk)
    return pl.pallas_call(
        body,
        out_shape=jax.ShapeDtypeStruct((do, s, s, cout), _F32),
    )(p, w27, b)


def _conv27(h_ref, w_ref, mm):
    def tap(t, acc):
        kd, kh = t // 3, t % 3
        for kw in range(3):
            xs = h_ref[pl.ds(kd, 16), pl.ds(kh, 16),
                       kw:kw + 16, :].reshape(4096, 128)
            acc = acc + mm(xs, w_ref[3 * t + kw])
        return acc
    return lax.fori_loop(0, 9, tap, jnp.zeros((4096, 128), _F32))


def _e3q_body(h_ref, w_ref, b_ref, qw_ref, qb_ref, o_ref):
    h3 = _conv27(h_ref, w_ref, _mmt) + b_ref[...]
    o_ref[...] = _mmt(h3, qw_ref[...]) + qb_ref[...]


def _e3q(h2p, wd, b3, qwt, qb):
    return pl.pallas_call(
        _e3q_body,
        out_shape=jax.ShapeDtypeStruct((4096, 128), _F32),
    )(h2p, wd, b3, qwt, qb)


def _vq_body(z_ref, ct_ref, cb_ref, pqw_ref, pqb_ref, ind_ref, cb2_ref,
             m_s, a_s):
    k = pl.program_id(0)
    cb = cb_ref[...]
    cb2_ref[...] = _mm(cb, pqw_ref[...]) + pqb_ref[...]
    cn = jnp.sum(cb * cb, axis=1)
    ms, as_ = [], []
    for rc in range(4):
        z = z_ref[rc * 1024:(rc + 1) * 1024, :]
        zn = jnp.sum(z * z, axis=1, keepdims=True)
        zc = _mm(z, ct_ref[...])
        d2 = (zn + cn[None, :]) - 2.0 * zc
        ms.append(jnp.min(d2, axis=1, keepdims=True))
        as_.append(jnp.argmin(d2, axis=1).astype(jnp.int32).reshape(1024, 1))
    m = jnp.concatenate(ms, axis=0)
    a = jnp.concatenate(as_, axis=0) + k * 1024

    @pl.when(k == 0)
    def _():
        m_s[...] = m
        a_s[...] = a

    @pl.when(k > 0)
    def _():
        better = m < m_s[...]
        m_s[...] = jnp.where(better, m, m_s[...])
        a_s[...] = jnp.where(better, a, a_s[...])

    ind_ref[...] = a_s[...]


def _vq(z, cbt, cb, pqwt, pqb):
    return pl.pallas_call(
        _vq_body,
        grid=(8,),
        in_specs=[
            pl.BlockSpec((4096, 128), lambda k: (0, 0)),
            pl.BlockSpec((128, 1024), lambda k: (0, k)),
            pl.BlockSpec((1024, 128), lambda k: (k, 0)),
            pl.BlockSpec((128, 128), lambda k: (0, 0)),
            pl.BlockSpec((1, 128), lambda k: (0, 0)),
        ],
        out_specs=[
            pl.BlockSpec((4096, 1), lambda k: (0, 0)),
            pl.BlockSpec((1024, 128), lambda k: (k, 0)),
        ],
        out_shape=[
            jax.ShapeDtypeStruct((4096, 1), jnp.int32),
            jax.ShapeDtypeStruct((8192, 128), _F32),
        ],
        scratch_shapes=[
            pltpu.VMEM((4096, 1), _F32),
            pltpu.VMEM((4096, 1), jnp.int32),
        ],
    )(z, cbt, cb, pqwt, pqb)


def _sc_gather(table, idx):
    info = plsc.get_sparse_core_info()
    nw = info.num_cores * info.num_subcores
    bpw = 4096 // nw
    mesh = plsc.VectorSubcoreMesh(core_axis_name="c", subcore_axis_name="s")

    @functools.partial(
        pl.kernel, mesh=mesh,
        out_type=jax.ShapeDtypeStruct((4096, 256), _F32),
        scratch_types=[
            pltpu.VMEM((bpw,), jnp.int32),
            pltpu.VMEM((bpw, 256), _F32),
            pltpu.SemaphoreType.DMA,
        ],
    )
    def k(table_hbm, idx_hbm, out_hbm, idx_v, rows_v, sem):
        wid = lax.axis_index("s") * info.num_cores + lax.axis_index("c")
        base = wid * bpw
        pltpu.sync_copy(idx_hbm.at[pl.ds(base, bpw)], idx_v)
        pltpu.async_copy(table_hbm.at[idx_v], rows_v, sem).wait()
        pltpu.sync_copy(rows_v, out_hbm.at[pl.ds(base, bpw)])

    return k(table, idx)


def _diff_body(z_ref, zq_ref, o_ref):
    d = zq_ref[...] - z_ref[...]
    m = jnp.sum(d * d) * _F32(1.0 / (4096.0 * 128.0))
    o_ref[...] = (m + _F32(0.25) * m).reshape(1, 1)


def _diff(z, zq):
    return pl.pallas_call(
        _diff_body,
        out_shape=jax.ShapeDtypeStruct((1, 1), _F32),
    )(z, zq)


def _d0_body(h_ref, w_ref, b_ref, o_ref):
    o_ref[...] = _swish(_conv27(h_ref, w_ref, _mm) + b_ref[...])


def _d0(tp, wg, b):
    return pl.pallas_call(
        _d0_body,
        out_shape=jax.ShapeDtypeStruct((4096, 128), _F32),
    )(tp, wg, b)


def _d1_body(g_ref, w_ref, b_ref, o_ref):
    for p, (pd, ph, pw) in enumerate(itertools.product(range(2), repeat=3)):
        def tap(t, acc):
            dd, dh = t // 2, t % 2
            for dw in range(2):
                xs = g_ref[pl.ds(pd + dd, 16), pl.ds(ph + dh, 16),
                           pw + dw:pw + dw + 16, :].reshape(4096, 128)
                acc = acc + _mm(xs, w_ref[p, 2 * t + dw])
            return acc
        acc = lax.fori_loop(0, 4, tap, jnp.zeros((4096, 64), _F32))
        o_ref[:, p * 64:(p + 1) * 64] = _swish(acc + b_ref[...])


def _d1(gp, wh, b):
    return pl.pallas_call(
        _d1_body,
        out_shape=jax.ShapeDtypeStruct((4096, 512), _F32),
    )(gp, wh, b)


def _d2_body(u_ref, w_ref, b_ref, o_ref):
    for p, (pd, ph, pw) in enumerate(itertools.product(range(2), repeat=3)):
        def tap(dh, acc):
            for dw in range(2):
                xs = u_ref[0, pl.ds(pd, 4), pl.ds(ph + dh, 32),
                           pw + dw:pw + dw + 32, :].reshape(4096, 128)
                acc = acc + _mm(xs, w_ref[p, 2 * dh + dw])
            return acc
        acc = lax.fori_loop(0, 2, tap, jnp.zeros((4096, 32), _F32))
        o_ref[:, p * 32:(p + 1) * 32] = _swish(acc + b_ref[...])


def _d2(ud2q, wi, b):
    return pl.pallas_call(
        _d2_body,
        grid=(8,),
        in_specs=[
            pl.BlockSpec((1, 5, 34, 34, 128), lambda q: (q, 0, 0, 0, 0)),
            pl.BlockSpec((8, 4, 128, 32), lambda q: (0, 0, 0, 0)),
            pl.BlockSpec((1, 32), lambda q: (0, 0)),
        ],
        out_specs=pl.BlockSpec((4096, 256), lambda q: (q, 0)),
        out_shape=jax.ShapeDtypeStruct((32768, 256), _F32),
    )(ud2q, wi, b)


def _d3_body(x_ref, b9_ref, b_ref, o_ref):
    def tap(kd, acc):
        for kh in range(3):
            xs = x_ref[0, pl.ds(kd, 16), kh:kh + 64, :].reshape(1024, 2112)
            acc = acc + _mm(xs, b9_ref[3 * kd + kh])
        return acc
    acc = lax.fori_loop(0, 3, tap, jnp.zeros((1024, 64), _F32))
    o_ref[...] = acc + b_ref[...]


def _d3(ufp4, b9, b):
    return pl.pallas_call(
        _d3_body,
        grid=(4,),
        in_specs=[
            pl.BlockSpec((1, 18, 66, 2112), lambda c: (c, 0, 0, 0)),
            pl.BlockSpec((9, 2112, 64), lambda c: (0, 0, 0)),
            pl.BlockSpec((1, 1), lambda c: (0, 0)),
        ],
        out_specs=pl.BlockSpec((1024, 64), lambda c: (c, 0)),
        out_shape=jax.ShapeDtypeStruct((4096, 64), _F32),
    )(ufp4, b9, b)


def _phase_map2():
    m = jnp.zeros((2, 2, 3), _F32)
    m = m.at[0, 0, 0].set(1.0).at[1, 0, 1].set(1.0).at[0, 1, 2].set(1.0)
    return m


def _phase_map_up():
    m = jnp.zeros((2, 2, 3), _F32)
    m = m.at[0, 0, 0].set(1.0)
    m = m.at[0, 1, 1].set(1.0).at[0, 1, 2].set(1.0)
    m = m.at[1, 0, 0].set(1.0).at[1, 0, 1].set(1.0)
    m = m.at[1, 1, 2].set(1.0)
    return m


def _s2d(x, n, c):
    d = x.shape[0] // 2
    return (x.reshape(d, 2, d, 2, d, 2, c)
            .transpose(0, 2, 4, 1, 3, 5, 6)
            .reshape(d, d, d, 8 * c))


def kernel(input, e_w0, e_b0, e_w1, e_b1, e_w2, e_b2, e_w3, e_b3, q_w, q_b,
           codebook, pq_w, pq_b, d_w0, d_b0, d_w1, d_b1, d_w2, d_b2,
           d_w3, d_b3):
    f32 = _F32
    eyes = jnp.stack([jnp.eye(66, 64, k=-kw, dtype=f32) for kw in range(3)])

    a9 = jnp.einsum('kpw,odhk->dhpwo', eyes, e_w0[:, 0].transpose(0, 1, 2, 3),
                    precision=_HI).reshape(9, 66, 2048)
    b2048 = jnp.tile(e_b0, (64,))[None, :]
    xp = jnp.pad(input.reshape(64, 64, 64), 1)
    h0 = _e0(xp, a9, b2048).reshape(64, 64, 64, 32)

    wb = jnp.transpose(e_w1, (2, 3, 4, 1, 0)).reshape(864, 64)
    h0p = jnp.pad(h0, ((1, 1), (1, 1), (1, 1), (0, 0)))
    p2 = _s2d(h0p, 33, 32)
    h1a = _s2_conv(p2[0:17], wb, e_b1[None], 16, 32, 32, 64, 2)
    h1b = _s2_conv(p2[16:33], wb, e_b1[None], 16, 32, 32, 64, 2)
    h1 = jnp.concatenate([h1a, h1b], axis=0)

    wc = jnp.transpose(e_w2, (2, 3, 4, 1, 0)).reshape(1728, 128)
    h1p = jnp.pad(h1, ((1, 1), (1, 1), (1, 1), (0, 0)))
    p3 = _s2d(h1p, 17, 64)
    h2 = _s2_conv(p3, wc, e_b2[None], 16, 16, 64, 128, 4)

    wd = jnp.transpose(e_w3, (2, 3, 4, 1, 0)).reshape(27, 128, 128)
    qwt = q_w.reshape(128, 128).T
    h2p = jnp.pad(h2, ((1, 1), (1, 1), (1, 1), (0, 0)))
    z = _e3q(h2p, wd, e_b3[None], qwt, q_b[None])

    pqwt = pq_w.reshape(128, 128).T
    ind2d, cb2 = _vq(z, codebook.T, codebook, pqwt, pq_b[None])
    cball = jnp.concatenate([codebook, cb2], axis=1)
    zqt = _sc_gather(cball, ind2d.reshape(4096))
    z_q = zqt[:, :128]
    t = zqt[:, 128:]

    diff = _diff(z, z_q).reshape(())

    wg = jnp.transpose(d_w0, (2, 3, 4, 1, 0)).reshape(27, 128, 128)
    tp = jnp.pad(t.reshape(16, 16, 16, 128), ((1, 1), (1, 1), (1, 1), (0, 0)))
    g = _d0(tp, wg, d_b0[None])

    mu = _phase_map_up()
    wh = jnp.einsum('xak,ybl,zcm,oiklm->xyzabcio', mu, mu, mu, d_w1,
                    precision=_HI).reshape(8, 8, 128, 64)
    gp = jnp.pad(g.reshape(16, 16, 16, 128), ((1, 1), (1, 1), (1, 1), (0, 0)))
    hh = _d1(gp, wh, d_b1[None])
    u = (hh.reshape(16, 16, 16, 2, 2, 2, 64)
         .transpose(0, 3, 1, 4, 2, 5, 6).reshape(32, 32, 32, 64))

    wi = jnp.einsum('xak,ybl,zcm,oiklm->xyzbcaio', mu, mu, mu, d_w2,
                    precision=_HI).reshape(8, 4, 128, 32)
    up = jnp.pad(u, ((1, 1), (1, 1), (1, 1), (0, 0)))
    ud2 = jnp.concatenate([up[0:33], up[1:34]], axis=-1)
    ud2q = jnp.stack([ud2[4 * e:4 * e + 5] for e in range(8)])
    si = _d2(ud2q, wi, d_b2[None])
    uf = (si.reshape(32, 32, 32, 2, 2, 2, 32)
          .transpose(0, 3, 1, 4, 2, 5, 6).reshape(64, 64, 64, 32))

    b9 = jnp.einsum('kpw,idhk->dhpiw', eyes, d_w3[0],
                    precision=_HI).reshape(9, 2112, 64)
    ufp = jnp.pad(uf, ((1, 1), (1, 1), (1, 1), (0, 0))).reshape(66, 66, 2112)
    ufp4 = jnp.stack([ufp[16 * c:16 * c + 18] for c in range(4)])
    dec = _d3(ufp4, b9, d_b3.reshape(1, 1))

    return dec.reshape(1, 1, 64, 64, 64), diff


# speedup vs baseline: 1.0581x; 1.0581x over previous
import functools
import itertools

import jax
import jax.numpy as jnp
from jax import lax
from jax.experimental import pallas as pl
from jax.experimental.pallas import tpu as pltpu
from jax.experimental.pallas import tpu_sc as plsc

_F32 = jnp.float32
_HI = lax.Precision.HIGHEST


def _mm(a, b, prec=None):
    return lax.dot_general(
        a, b, dimension_numbers=(((a.ndim - 1,), (0,)), ((), ())),
        preferred_element_type=_F32, precision=prec)


def _mmt(a, b):
    tb = jnp.bfloat16
    return _mm(a.astype(tb).astype(_F32), b.astype(tb).astype(_F32))


def _mm4(a, b):
    tb = jnp.bfloat16
    ah = a.astype(tb).astype(_F32)
    al = (a - ah).astype(tb).astype(_F32)
    bh = b.astype(tb).astype(_F32)
    bl = (b - bh).astype(tb).astype(_F32)
    return _mm(ah, bh) + (_mm(ah, bl) + (_mm(al, bh) + _mm(al, bl)))


def _swish(x):
    return x * jax.nn.sigmoid(x)


def _e0_body(x_ref, a_ref, b_ref, o_ref):
    c = pl.program_id(0)
    def tap(kd, acc):
        for kh in range(3):
            xs = x_ref[pl.ds(kd + 16 * c, 16), kh:kh + 64, :].reshape(1024, 66)
            acc = acc + _mmt(xs, a_ref[3 * kd + kh])
        return acc
    acc = lax.fori_loop(0, 3, tap, jnp.zeros((1024, 2048), _F32))
    o_ref[...] = _swish(acc + b_ref[...])


def _e0(xp, a9, b2048):
    return pl.pallas_call(
        _e0_body,
        grid=(4,),
        in_specs=[
            pl.BlockSpec((66, 66, 66), lambda c: (0, 0, 0)),
            pl.BlockSpec((9, 66, 2048), lambda c: (0, 0, 0)),
            pl.BlockSpec((1, 2048), lambda c: (0, 0)),
        ],
        out_specs=pl.BlockSpec((1024, 2048), lambda c: (c, 0)),
        out_shape=jax.ShapeDtypeStruct((4096, 2048), _F32),
    )(xp, a9, b2048)


_TAPMAP = {0: (0, 0), 1: (1, 0), 2: (0, 1)}


def _s2_body(p_ref, w_ref, b_ref, o_ref, *, do, s, cin, cout, dchunk):
    for c0 in range(0, do, dchunk):
        cols = []
        for kd in range(3):
            pd, td = _TAPMAP[kd]
            for kh in range(3):
                ph, th = _TAPMAP[kh]
                for kw in range(3):
                    pw, tw = _TAPMAP[kw]
                    pc = (pd * 2 + ph) * 2 + pw
                    cols.append(
                        p_ref[td + c0:td + c0 + dchunk, th:th + s,
                              tw:tw + s, pc * cin:(pc + 1) * cin]
                        .reshape(dchunk * s * s, cin))
        xcat = jnp.concatenate(cols, axis=1)
        acc = _mmt(xcat, w_ref[...])
        o_ref[c0:c0 + dchunk] = _swish(acc + b_ref[...]).reshape(
            dchunk, s, s, cout)


def _s2_conv(p, w27, b, do, s, cin, cout, dchunk):
    body = functools.partial(_s2_body, do=do, s=s, cin=cin, cout=cout,
                             dchunk=dchunk)
    return pl.pallas_call(
        body,
        out_shape=jax.ShapeDtypeStruct((do, s, s, cout), _F32),
    )(p, w27, b)


def _conv27(h_ref, w_ref, mm):
    def tap(t, acc):
        kd, kh = t // 3, t % 3
        for kw in range(3):
            xs = h_ref[pl.ds(kd, 16), pl.ds(kh, 16),
                       kw:kw + 16, :].reshape(4096, 128)
            acc = acc + mm(xs, w_ref[3 * t + kw])
        return acc
    return lax.fori_loop(0, 9, tap, jnp.zeros((4096, 128), _F32))


def _e3q_body(h_ref, w_ref, b_ref, qw_ref, qb_ref, o_ref):
    h3 = _conv27(h_ref, w_ref, _mmt) + b_ref[...]
    o_ref[...] = _mmt(h3, qw_ref[...]) + qb_ref[...]


def _e3q(h2p, wd, b3, qwt, qb):
    return pl.pallas_call(
        _e3q_body,
        out_shape=jax.ShapeDtypeStruct((4096, 128), _F32),
    )(h2p, wd, b3, qwt, qb)


def _vq_body(z_ref, ct_ref, cb_ref, pqw_ref, pqb_ref, ind_ref, cb2_ref,
             m_s, a_s):
    k = pl.program_id(0)
    cb = cb_ref[...]
    cb2_ref[...] = _mm(cb, pqw_ref[...]) + pqb_ref[...]
    cn = jnp.sum(cb * cb, axis=1)
    ms, as_ = [], []
    for rc in range(4):
        z = z_ref[rc * 1024:(rc + 1) * 1024, :]
        zn = jnp.sum(z * z, axis=1, keepdims=True)
        zc = _mm(z, ct_ref[...])
        d2 = (zn + cn[None, :]) - 2.0 * zc
        ms.append(jnp.min(d2, axis=1, keepdims=True))
        as_.append(jnp.argmin(d2, axis=1).astype(jnp.int32).reshape(1024, 1))
    m = jnp.concatenate(ms, axis=0)
    a = jnp.concatenate(as_, axis=0) + k * 1024

    @pl.when(k == 0)
    def _():
        m_s[...] = m
        a_s[...] = a

    @pl.when(k > 0)
    def _():
        better = m < m_s[...]
        m_s[...] = jnp.where(better, m, m_s[...])
        a_s[...] = jnp.where(better, a, a_s[...])

    ind_ref[...] = a_s[...]


def _vq(z, cbt, cb, pqwt, pqb):
    return pl.pallas_call(
        _vq_body,
        grid=(8,),
        in_specs=[
            pl.BlockSpec((4096, 128), lambda k: (0, 0)),
            pl.BlockSpec((128, 1024), lambda k: (0, k)),
            pl.BlockSpec((1024, 128), lambda k: (k, 0)),
            pl.BlockSpec((128, 128), lambda k: (0, 0)),
            pl.BlockSpec((1, 128), lambda k: (0, 0)),
        ],
        out_specs=[
            pl.BlockSpec((4096, 1), lambda k: (0, 0)),
            pl.BlockSpec((1024, 128), lambda k: (k, 0)),
        ],
        out_shape=[
            jax.ShapeDtypeStruct((4096, 1), jnp.int32),
            jax.ShapeDtypeStruct((8192, 128), _F32),
        ],
        scratch_shapes=[
            pltpu.VMEM((4096, 1), _F32),
            pltpu.VMEM((4096, 1), jnp.int32),
        ],
    )(z, cbt, cb, pqwt, pqb)


def _sc_gather(table, idx):
    info = plsc.get_sparse_core_info()
    nw = info.num_cores * info.num_subcores
    bpw = 4096 // nw
    mesh = plsc.VectorSubcoreMesh(core_axis_name="c", subcore_axis_name="s")

    @functools.partial(
        pl.kernel, mesh=mesh,
        out_type=jax.ShapeDtypeStruct((4096, 256), _F32),
        scratch_types=[
            pltpu.VMEM((bpw,), jnp.int32),
            pltpu.VMEM((bpw, 256), _F32),
            pltpu.SemaphoreType.DMA,
        ],
    )
    def k(table_hbm, idx_hbm, out_hbm, idx_v, rows_v, sem):
        wid = lax.axis_index("s") * info.num_cores + lax.axis_index("c")
        base = wid * bpw
        pltpu.sync_copy(idx_hbm.at[pl.ds(base, bpw)], idx_v)
        pltpu.async_copy(table_hbm.at[idx_v], rows_v, sem).wait()
        pltpu.sync_copy(rows_v, out_hbm.at[pl.ds(base, bpw)])

    return k(table, idx)


def _diff_body(z_ref, zq_ref, o_ref):
    d = zq_ref[...] - z_ref[...]
    m = jnp.sum(d * d) * _F32(1.0 / (4096.0 * 128.0))
    o_ref[...] = (m + _F32(0.25) * m).reshape(1, 1)


def _diff(z, zq):
    return pl.pallas_call(
        _diff_body,
        out_shape=jax.ShapeDtypeStruct((1, 1), _F32),
    )(z, zq)


def _d0_body(h_ref, w_ref, b_ref, o_ref):
    o_ref[...] = _swish(_conv27(h_ref, w_ref, _mm) + b_ref[...])


def _d0(tp, wg, b):
    return pl.pallas_call(
        _d0_body,
        out_shape=jax.ShapeDtypeStruct((4096, 128), _F32),
    )(tp, wg, b)


def _d1_body(g_ref, w_ref, b_ref, o_ref):
    for p, (pd, ph, pw) in enumerate(itertools.product(range(2), repeat=3)):
        def tap(t, acc):
            dd, dh = t // 2, t % 2
            for dw in range(2):
                xs = g_ref[pl.ds(pd + dd, 16), pl.ds(ph + dh, 16),
                           pw + dw:pw + dw + 16, :].reshape(4096, 128)
                acc = acc + _mm(xs, w_ref[p, 2 * t + dw])
            return acc
        acc = lax.fori_loop(0, 4, tap, jnp.zeros((4096, 64), _F32))
        o_ref[:, p * 64:(p + 1) * 64] = _swish(acc + b_ref[...])


def _d1(gp, wh, b):
    return pl.pallas_call(
        _d1_body,
        out_shape=jax.ShapeDtypeStruct((4096, 512), _F32),
    )(gp, wh, b)


def _d2_body(u_ref, w_ref, b_ref, o_ref):
    q = pl.program_id(0)
    for p, (pd, ph, pw) in enumerate(itertools.product(range(2), repeat=3)):
        def tap(dh, acc):
            for dw in range(2):
                xs = u_ref[pl.ds(4 * q + pd, 4), pl.ds(ph + dh, 32),
                           pw + dw:pw + dw + 32, :].reshape(4096, 128)
                acc = acc + _mm(xs, w_ref[p, 2 * dh + dw])
            return acc
        acc = lax.fori_loop(0, 2, tap, jnp.zeros((4096, 32), _F32))
        o_ref[:, p * 32:(p + 1) * 32] = _swish(acc + b_ref[...])


def _d2(ud2, wi, b):
    return pl.pallas_call(
        _d2_body,
        grid=(8,),
        in_specs=[
            pl.BlockSpec((33, 34, 34, 128), lambda q: (0, 0, 0, 0)),
            pl.BlockSpec((8, 4, 128, 32), lambda q: (0, 0, 0, 0)),
            pl.BlockSpec((1, 32), lambda q: (0, 0)),
        ],
        out_specs=pl.BlockSpec((4096, 256), lambda q: (q, 0)),
        out_shape=jax.ShapeDtypeStruct((32768, 256), _F32),
    )(ud2, wi, b)


def _d3_body(x_ref, b9_ref, b_ref, o_ref):
    c = pl.program_id(0)
    def tap(kd, acc):
        for kh in range(3):
            xs = x_ref[pl.ds(kd + 8 * c, 8), kh:kh + 64, :].reshape(512, 2112)
            acc = acc + _mm(xs, b9_ref[3 * kd + kh])
        return acc
    acc = lax.fori_loop(0, 3, tap, jnp.zeros((512, 64), _F32))
    o_ref[...] = acc + b_ref[...]


def _d3(ufp, b9, b):
    return pl.pallas_call(
        _d3_body,
        grid=(8,),
        in_specs=[
            pl.BlockSpec((66, 66, 2112), lambda c: (0, 0, 0)),
            pl.BlockSpec((9, 2112, 64), lambda c: (0, 0, 0)),
            pl.BlockSpec((1, 1), lambda c: (0, 0)),
        ],
        out_specs=pl.BlockSpec((512, 64), lambda c: (c, 0)),
        out_shape=jax.ShapeDtypeStruct((4096, 64), _F32),
    )(ufp, b9, b)


def _phase_map2():
    m = jnp.zeros((2, 2, 3), _F32)
    m = m.at[0, 0, 0].set(1.0).at[1, 0, 1].set(1.0).at[0, 1, 2].set(1.0)
    return m


def _phase_map_up():
    m = jnp.zeros((2, 2, 3), _F32)
    m = m.at[0, 0, 0].set(1.0)
    m = m.at[0, 1, 1].set(1.0).at[0, 1, 2].set(1.0)
    m = m.at[1, 0, 0].set(1.0).at[1, 0, 1].set(1.0)
    m = m.at[1, 1, 2].set(1.0)
    return m


def _s2d(x, n, c):
    d = x.shape[0] // 2
    return (x.reshape(d, 2, d, 2, d, 2, c)
            .transpose(0, 2, 4, 1, 3, 5, 6)
            .reshape(d, d, d, 8 * c))


def kernel(input, e_w0, e_b0, e_w1, e_b1, e_w2, e_b2, e_w3, e_b3, q_w, q_b,
           codebook, pq_w, pq_b, d_w0, d_b0, d_w1, d_b1, d_w2, d_b2,
           d_w3, d_b3):
    f32 = _F32
    eyes = jnp.stack([jnp.eye(66, 64, k=-kw, dtype=f32) for kw in range(3)])

    a9 = jnp.einsum('kpw,odhk->dhpwo', eyes, e_w0[:, 0].transpose(0, 1, 2, 3),
                    precision=_HI).reshape(9, 66, 2048)
    b2048 = jnp.tile(e_b0, (64,))[None, :]
    xp = jnp.pad(input.reshape(64, 64, 64), 1)
    h0 = _e0(xp, a9, b2048).reshape(64, 64, 64, 32)

    wb = jnp.transpose(e_w1, (2, 3, 4, 1, 0)).reshape(864, 64)
    h0p = jnp.pad(h0, ((1, 1), (1, 1), (1, 1), (0, 0)))
    p2 = _s2d(h0p, 33, 32)
    h1a = _s2_conv(p2[0:17], wb, e_b1[None], 16, 32, 32, 64, 2)
    h1b = _s2_conv(p2[16:33], wb, e_b1[None], 16, 32, 32, 64, 2)
    h1 = jnp.concatenate([h1a, h1b], axis=0)

    wc = jnp.transpose(e_w2, (2, 3, 4, 1, 0)).reshape(1728, 128)
    h1p = jnp.pad(h1, ((1, 1), (1, 1), (1, 1), (0, 0)))
    p3 = _s2d(h1p, 17, 64)
    h2 = _s2_conv(p3, wc, e_b2[None], 16, 16, 64, 128, 4)

    wd = jnp.transpose(e_w3, (2, 3, 4, 1, 0)).reshape(27, 128, 128)
    qwt = q_w.reshape(128, 128).T
    h2p = jnp.pad(h2, ((1, 1), (1, 1), (1, 1), (0, 0)))
    z = _e3q(h2p, wd, e_b3[None], qwt, q_b[None])

    pqwt = pq_w.reshape(128, 128).T
    ind2d, cb2 = _vq(z, codebook.T, codebook, pqwt, pq_b[None])
    cball = jnp.concatenate([codebook, cb2], axis=1)
    zqt = _sc_gather(cball, ind2d.reshape(4096))
    z_q = zqt[:, :128]
    t = zqt[:, 128:]

    diff = _diff(z, z_q).reshape(())

    wg = jnp.transpose(d_w0, (2, 3, 4, 1, 0)).reshape(27, 128, 128)
    tp = jnp.pad(t.reshape(16, 16, 16, 128), ((1, 1), (1, 1), (1, 1), (0, 0)))
    g = _d0(tp, wg, d_b0[None])

    mu = _phase_map_up()
    wh = jnp.einsum('xak,ybl,zcm,oiklm->xyzabcio', mu, mu, mu, d_w1,
                    precision=_HI).reshape(8, 8, 128, 64)
    gp = jnp.pad(g.reshape(16, 16, 16, 128), ((1, 1), (1, 1), (1, 1), (0, 0)))
    hh = _d1(gp, wh, d_b1[None])
    u = (hh.reshape(16, 16, 16, 2, 2, 2, 64)
         .transpose(0, 3, 1, 4, 2, 5, 6).reshape(32, 32, 32, 64))

    wi = jnp.einsum('xak,ybl,zcm,oiklm->xyzbcaio', mu, mu, mu, d_w2,
                    precision=_HI).reshape(8, 4, 128, 32)
    up = jnp.pad(u, ((1, 1), (1, 1), (1, 1), (0, 0)))
    ud2 = jnp.concatenate([up[0:33], up[1:34]], axis=-1)
    si = _d2(ud2, wi, d_b2[None])
    uf = (si.reshape(32, 32, 32, 2, 2, 2, 32)
          .transpose(0, 3, 1, 4, 2, 5, 6).reshape(64, 64, 64, 32))

    b9 = jnp.einsum('kpw,idhk->dhpiw', eyes, d_w3[0],
                    precision=_HI).reshape(9, 2112, 64)
    ufp = jnp.pad(uf, ((1, 1), (1, 1), (1, 1), (0, 0))).reshape(66, 66, 2112)
    dec = _d3(ufp, b9, d_b3.reshape(1, 1))

    return dec.reshape(1, 1, 64, 64, 64), diff
